# baseline (speedup 1.0000x reference)
import functools
import math

import jax
import jax.numpy as jnp
import numpy as np
from jax import lax
from jax.experimental import pallas as pl
from jax.experimental.pallas import tpu as pltpu
from jax.experimental.pallas import tpu_sc as plsc

D_MODEL = 1024
MAX_LEN = 5000
PE_DIM = D_MODEL // 4


def _pe_table() -> np.ndarray:
    position = np.arange(MAX_LEN, dtype=np.float32)[:, None]
    div_term = np.exp(
        np.arange(0, PE_DIM, 2).astype(np.float32) * (-math.log(10000.0) / PE_DIM)
    )
    pe = np.zeros((MAX_LEN, PE_DIM), dtype=np.float32)
    pe[:, 0::2] = np.sin(position * div_term)
    pe[:, 1::2] = np.cos(position * div_term)
    return pe


_PE = _pe_table()

_OCHUNK = 64


def _sc_gather(pe, idx, n_out):
    info = plsc.get_sparse_core_info()
    nw = info.num_cores * info.num_subcores
    n_j = n_out // _OCHUNK
    assert n_j * _OCHUNK == n_out
    n_chunks = 4 * n_j

    mesh = plsc.VectorSubcoreMesh(core_axis_name="c", subcore_axis_name="s")

    nbuf = 4

    @functools.partial(
        pl.kernel,
        mesh=mesh,
        out_type=jax.ShapeDtypeStruct((n_out, D_MODEL), jnp.float32),
        scratch_types=(
            [pltpu.VMEM((_OCHUNK,), jnp.int32)] * nbuf
            + [pltpu.VMEM((_OCHUNK, PE_DIM), jnp.float32)] * nbuf
            + [pltpu.SemaphoreType.DMA] * (2 * nbuf)
        ),
    )
    def k(pe_hbm, idx_hbm, out_hbm, *scr):
        idx_vs = scr[:nbuf]
        rows_vs = scr[nbuf : 2 * nbuf]
        gsem = scr[2 * nbuf : 3 * nbuf]
        wsem = scr[3 * nbuf : 4 * nbuf]
        wid = lax.axis_index("s") * info.num_cores + lax.axis_index("c")
        n_mine = (n_chunks - wid + nw - 1) // nw

        def out_slice(u):
            chunk = wid + u * nw
            kk = chunk // n_j
            j = chunk % n_j
            return out_hbm.at[
                pl.ds(j * _OCHUNK, _OCHUNK), pl.ds(kk * PE_DIM, PE_DIM)
            ]

        def start(b, u):
            @pl.when(u < n_mine)
            def _():
                chunk = wid + u * nw
                kk = chunk // n_j
                j = chunk % n_j
                pltpu.sync_copy(
                    idx_hbm.at[pl.ds(kk * n_out + j * _OCHUNK, _OCHUNK)], idx_vs[b]
                )
                pltpu.async_copy(pe_hbm.at[idx_vs[b]], rows_vs[b], gsem[b])

        def flush(b, u):
            @pl.when(u < n_mine)
            def _():
                pltpu.make_async_copy(
                    pe_hbm.at[idx_vs[b]], rows_vs[b], gsem[b]
                ).wait()
                pltpu.async_copy(rows_vs[b], out_slice(u), wsem[b])

        def wdrain(b, u):
            @pl.when(u < n_mine)
            def _():
                pltpu.make_async_copy(rows_vs[b], out_slice(u), wsem[b]).wait()

        for b in range(nbuf):
            start(b, b)

        def body(t, carry):
            u0 = nbuf * t
            for b in range(nbuf):
                flush(b, u0 + b)
            for b in range(nbuf):
                wdrain(b, u0 + b)
                start(b, u0 + nbuf + b)
            return carry

        n_rounds = ((n_chunks + nw - 1) // nw + nbuf - 1) // nbuf
        lax.fori_loop(0, n_rounds, body, 0)

    return k(pe, idx)


def _tc_add_group(seq_t, emb3_g, c0, accs):
    s, b, d = seq_t.shape
    sg = emb3_g.shape[0]
    acc_spec = pl.BlockSpec((1, b, d), lambda c: (c + c0, 0, 0))
    emb_spec = pl.BlockSpec((1, b, d), lambda c: (c, 0, 0))
    any_spec = pl.BlockSpec(memory_space=pl.ANY)

    def body(seq_ref, emb_ref, *rest):
        out_ref, embout_ref = rest[-2:]
        e = emb_ref[...]
        out_ref[...] = seq_ref[...] + e
        embout_ref[...] = e

    operands = [seq_t, emb3_g]
    in_specs = [acc_spec, emb_spec]
    aliases = {}
    if accs is not None:
        operands += list(accs)
        in_specs += [any_spec, any_spec]
        aliases = {2: 0, 3: 1}

    return pl.pallas_call(
        body,
        grid=(sg,),
        in_specs=in_specs,
        out_specs=[acc_spec, acc_spec],
        out_shape=[
            jax.ShapeDtypeStruct((s, b, d), jnp.float32),
            jax.ShapeDtypeStruct((s, b, d), jnp.float32),
        ],
        input_output_aliases=aliases,
    )(*operands)


_BOUNDS = (0, 120, 210, 257)


def kernel(seq, coords, seq_id):
    b, s, d = seq.shape

    ii = jnp.concatenate(
        [coords.astype(jnp.int32), seq_id[..., None].astype(jnp.int32)], axis=-1
    )
    ii = jnp.clip(ii, 0, MAX_LEN - 1)
    ii = jnp.pad(ii, ((0, 0), (1, 0), (0, 0)))
    ii_t = jnp.transpose(ii, (2, 1, 0))

    pe = jnp.asarray(_PE)
    seq_t = jnp.transpose(seq, (1, 0, 2))

    accs = None
    for g in range(len(_BOUNDS) - 1):
        c0, c1 = _BOUNDS[g], _BOUNDS[g + 1]
        sg = c1 - c0
        idx_g = ii_t[:, c0:c1, :].reshape(4 * sg * b)
        emb_g = _sc_gather(pe, idx_g, sg * b)
        accs = _tc_add_group(seq_t, emb_g.reshape(sg, b, d), c0, accs)

    out = jnp.transpose(accs[0], (1, 0, 2))
    emb = jnp.transpose(accs[1], (1, 0, 2))
    return (out, emb)


# speedup vs baseline: 1.0310x; 1.0310x over previous
import functools
import math

import jax
import jax.numpy as jnp
import numpy as np
from jax import lax
from jax.experimental import pallas as pl
from jax.experimental.pallas import tpu as pltpu
from jax.experimental.pallas import tpu_sc as plsc

D_MODEL = 1024
MAX_LEN = 5000
PE_DIM = D_MODEL // 4


def _pe_table() -> np.ndarray:
    position = np.arange(MAX_LEN, dtype=np.float32)[:, None]
    div_term = np.exp(
        np.arange(0, PE_DIM, 2).astype(np.float32) * (-math.log(10000.0) / PE_DIM)
    )
    pe = np.zeros((MAX_LEN, PE_DIM), dtype=np.float32)
    pe[:, 0::2] = np.sin(position * div_term)
    pe[:, 1::2] = np.cos(position * div_term)
    return pe


_PE = _pe_table()

_OCHUNK = 64


def _sc_gather(pe, idx, n_out):
    info = plsc.get_sparse_core_info()
    nw = info.num_cores * info.num_subcores
    n_j = n_out // _OCHUNK
    assert n_j * _OCHUNK == n_out
    n_chunks = 4 * n_j

    mesh = plsc.VectorSubcoreMesh(core_axis_name="c", subcore_axis_name="s")

    nbuf = 4

    @functools.partial(
        pl.kernel,
        mesh=mesh,
        out_type=jax.ShapeDtypeStruct((n_out, D_MODEL), jnp.float32),
        scratch_types=(
            [pltpu.VMEM((_OCHUNK,), jnp.int32)] * nbuf
            + [pltpu.VMEM((_OCHUNK, PE_DIM), jnp.float32)] * nbuf
            + [pltpu.SemaphoreType.DMA] * (2 * nbuf)
        ),
    )
    def k(pe_hbm, idx_hbm, out_hbm, *scr):
        idx_vs = scr[:nbuf]
        rows_vs = scr[nbuf : 2 * nbuf]
        gsem = scr[2 * nbuf : 3 * nbuf]
        wsem = scr[3 * nbuf : 4 * nbuf]
        wid = lax.axis_index("s") * info.num_cores + lax.axis_index("c")
        n_mine = (n_chunks - wid + nw - 1) // nw

        def out_slice(u):
            chunk = wid + u * nw
            kk = chunk // n_j
            j = chunk % n_j
            return out_hbm.at[
                pl.ds(j * _OCHUNK, _OCHUNK), pl.ds(kk * PE_DIM, PE_DIM)
            ]

        def start(b, u):
            @pl.when(u < n_mine)
            def _():
                chunk = wid + u * nw
                kk = chunk // n_j
                j = chunk % n_j
                pltpu.sync_copy(
                    idx_hbm.at[pl.ds(kk * n_out + j * _OCHUNK, _OCHUNK)], idx_vs[b]
                )
                pltpu.async_copy(pe_hbm.at[idx_vs[b]], rows_vs[b], gsem[b])

        def flush(b, u):
            @pl.when(u < n_mine)
            def _():
                pltpu.make_async_copy(
                    pe_hbm.at[idx_vs[b]], rows_vs[b], gsem[b]
                ).wait()
                pltpu.async_copy(rows_vs[b], out_slice(u), wsem[b])

        def wdrain(b, u):
            @pl.when(u < n_mine)
            def _():
                pltpu.make_async_copy(rows_vs[b], out_slice(u), wsem[b]).wait()

        for b in range(nbuf):
            start(b, b)

        def body(t, carry):
            u0 = nbuf * t
            for b in range(nbuf):
                flush(b, u0 + b)
            for b in range(nbuf):
                wdrain(b, u0 + b)
                start(b, u0 + nbuf + b)
            return carry

        n_rounds = ((n_chunks + nw - 1) // nw + nbuf - 1) // nbuf
        lax.fori_loop(0, n_rounds, body, 0)

    return k(pe, idx)


def _tc_add_group(seq_t, emb3_g, c0, accs):
    s, b, d = seq_t.shape
    sg = emb3_g.shape[0]
    acc_spec = pl.BlockSpec((1, b, d), lambda c: (c + c0, 0, 0))
    emb_spec = pl.BlockSpec((1, b, d), lambda c: (c, 0, 0))
    any_spec = pl.BlockSpec(memory_space=pl.ANY)

    def body(seq_ref, emb_ref, *rest):
        out_ref, embout_ref = rest[-2:]
        e = emb_ref[...]
        out_ref[...] = seq_ref[...] + e
        embout_ref[...] = e

    operands = [seq_t, emb3_g]
    in_specs = [acc_spec, emb_spec]
    aliases = {}
    if accs is not None:
        operands += list(accs)
        in_specs += [any_spec, any_spec]
        aliases = {2: 0, 3: 1}

    return pl.pallas_call(
        body,
        grid=(sg,),
        in_specs=in_specs,
        out_specs=[acc_spec, acc_spec],
        out_shape=[
            jax.ShapeDtypeStruct((s, b, d), jnp.float32),
            jax.ShapeDtypeStruct((s, b, d), jnp.float32),
        ],
        input_output_aliases=aliases,
    )(*operands)


_BOUNDS = (0, 144, 257)


def kernel(seq, coords, seq_id):
    b, s, d = seq.shape

    ii = jnp.concatenate(
        [coords.astype(jnp.int32), seq_id[..., None].astype(jnp.int32)], axis=-1
    )
    ii = jnp.clip(ii, 0, MAX_LEN - 1)
    ii = jnp.pad(ii, ((0, 0), (1, 0), (0, 0)))
    ii_t = jnp.transpose(ii, (2, 1, 0))

    pe = jnp.asarray(_PE)
    seq_t = jnp.transpose(seq, (1, 0, 2))

    accs = None
    for g in range(len(_BOUNDS) - 1):
        c0, c1 = _BOUNDS[g], _BOUNDS[g + 1]
        sg = c1 - c0
        idx_g = ii_t[:, c0:c1, :].reshape(4 * sg * b)
        emb_g = _sc_gather(pe, idx_g, sg * b)
        accs = _tc_add_group(seq_t, emb_g.reshape(sg, b, d), c0, accs)

    out = jnp.transpose(accs[0], (1, 0, 2))
    emb = jnp.transpose(accs[1], (1, 0, 2))
    return (out, emb)
